# baseline (speedup 1.0000x reference)
import functools

import jax
import jax.numpy as jnp
from jax.experimental import pallas as pl


def _moe_slice(xb, wg, we_cat, e):
    nv = xb.shape[1]

    logits = jnp.dot(wg, xb, preferred_element_type=jnp.float32)
    m = jnp.max(logits, axis=0, keepdims=True)
    ex = jnp.exp(logits - m)
    g = ex / jnp.sum(ex, axis=0, keepdims=True)

    neg = jnp.float32(-jnp.inf)
    m1 = jnp.max(g, axis=0, keepdims=True)
    m2 = jnp.max(jnp.where(g < m1, g, neg), axis=0, keepdims=True)
    wd = jnp.where(g >= m2, g, 0.0)

    xb_b = xb.astype(jnp.bfloat16)
    wd_b = wd.astype(jnp.bfloat16)
    inlen = xb.shape[0]
    out = None
    base = 0
    for pp in range(e // 2):
        parts = [
            wd_b[2 * pp : 2 * pp + 1, :] * xb_b,
            wd_b[2 * pp + 1 : 2 * pp + 2, :] * xb_b,
        ]
        if pp == 0:
            parts.append(wd_b)
        xw_p = jnp.concatenate(parts, axis=0)
        width = xw_p.shape[0]
        y = jnp.dot(
            we_cat[:, base : base + width], xw_p,
            preferred_element_type=jnp.float32,
        )
        base += width
        out = y if out is None else out + y
    return out, g


def _moe_body(x_ref, wg_ref, we_ref, out_ref, gate_ref, *, nsteps, bt, e):
    s = pl.program_id(0)
    wg = wg_ref[...]
    we_cat = we_ref[...]

    gsum = None
    for bi in range(bt):
        out, g = _moe_slice(x_ref[bi], wg, we_cat, e)
        out_ref[bi] = out
        gsum = g if gsum is None else gsum + g

    @pl.when(s == 0)
    def _init():
        gate_ref[...] = jnp.zeros_like(gate_ref)

    gate_ref[...] += gsum

    @pl.when(s == nsteps - 1)
    def _fin():
        gate_ref[...] = gate_ref[...] * (1.0 / (nsteps * bt))


@jax.jit
def kernel(x, Wg, We, be):
    B, IN_LEN, NVARS = x.shape
    E, OUT_LEN, _ = We.shape
    BT = 4
    nsteps = B // BT

    wef = We.transpose(1, 0, 2).reshape(OUT_LEN, E * IN_LEN)
    we_cat = jnp.concatenate(
        [wef[:, : 2 * IN_LEN], be.T, wef[:, 2 * IN_LEN :]], axis=1
    ).astype(jnp.bfloat16)

    body = functools.partial(_moe_body, nsteps=nsteps, bt=BT, e=E)
    out, gate_sum = pl.pallas_call(
        body,
        grid=(nsteps,),
        in_specs=[
            pl.BlockSpec((BT, IN_LEN, NVARS), lambda s: (s, 0, 0)),
            pl.BlockSpec((E, IN_LEN), lambda s: (0, 0)),
            pl.BlockSpec((OUT_LEN, E * IN_LEN + E), lambda s: (0, 0)),
        ],
        out_specs=[
            pl.BlockSpec((BT, OUT_LEN, NVARS), lambda s: (s, 0, 0)),
            pl.BlockSpec((E, NVARS), lambda s: (0, 0)),
        ],
        out_shape=[
            jax.ShapeDtypeStruct((B, OUT_LEN, NVARS), x.dtype),
            jax.ShapeDtypeStruct((E, NVARS), jnp.float32),
        ],
    )(x, Wg, we_cat)

    gate_mean = gate_sum.T
    return (out, gate_mean)


# speedup vs baseline: 1.0442x; 1.0442x over previous
import functools

import jax
import jax.numpy as jnp
from jax.experimental import pallas as pl


def _moe_slice(xb, wg, we_cat, e):
    nv = xb.shape[1]

    logits = jnp.dot(wg, xb, preferred_element_type=jnp.float32)
    m = jnp.max(logits, axis=0, keepdims=True)
    ex = jnp.exp(logits - m)
    g = ex / jnp.sum(ex, axis=0, keepdims=True)

    neg = jnp.float32(-jnp.inf)
    m1 = jnp.max(g, axis=0, keepdims=True)
    m2 = jnp.max(jnp.where(g < m1, g, neg), axis=0, keepdims=True)
    wd = jnp.where(g >= m2, g, 0.0)

    xb_b = xb.astype(jnp.bfloat16)
    wd_b = wd.astype(jnp.bfloat16)
    inlen = xb.shape[0]
    ys = []
    base = 0
    for pp in range(e // 2):
        parts = [
            wd_b[2 * pp : 2 * pp + 1, :] * xb_b,
            wd_b[2 * pp + 1 : 2 * pp + 2, :] * xb_b,
        ]
        if pp == 0:
            parts.append(wd_b)
        xw_p = jnp.concatenate(parts, axis=0)
        width = xw_p.shape[0]
        ys.append(jnp.dot(
            we_cat[:, base : base + width], xw_p,
            preferred_element_type=jnp.float32,
        ))
        base += width
    out = (ys[0] + ys[1]) + (ys[2] + ys[3])
    return out, g


def _moe_body(x_ref, wg_ref, we_ref, out_ref, gate_ref, *, nsteps, bt, e):
    s = pl.program_id(0)
    wg = wg_ref[...]
    we_cat = we_ref[...]

    gsum = None
    for bi in range(bt):
        out, g = _moe_slice(x_ref[bi], wg, we_cat, e)
        out_ref[bi] = out
        gsum = g if gsum is None else gsum + g

    @pl.when(s == 0)
    def _init():
        gate_ref[...] = jnp.zeros_like(gate_ref)

    gate_ref[...] += gsum

    @pl.when(s == nsteps - 1)
    def _fin():
        gate_ref[...] = gate_ref[...] * (1.0 / (nsteps * bt))


@jax.jit
def kernel(x, Wg, We, be):
    B, IN_LEN, NVARS = x.shape
    E, OUT_LEN, _ = We.shape
    BT = 8
    nsteps = B // BT

    wef = We.transpose(1, 0, 2).reshape(OUT_LEN, E * IN_LEN)
    we_cat = jnp.concatenate(
        [wef[:, : 2 * IN_LEN], be.T, wef[:, 2 * IN_LEN :]], axis=1
    ).astype(jnp.bfloat16)

    body = functools.partial(_moe_body, nsteps=nsteps, bt=BT, e=E)
    out, gate_sum = pl.pallas_call(
        body,
        grid=(nsteps,),
        in_specs=[
            pl.BlockSpec((BT, IN_LEN, NVARS), lambda s: (s, 0, 0)),
            pl.BlockSpec((E, IN_LEN), lambda s: (0, 0)),
            pl.BlockSpec((OUT_LEN, E * IN_LEN + E), lambda s: (0, 0)),
        ],
        out_specs=[
            pl.BlockSpec((BT, OUT_LEN, NVARS), lambda s: (s, 0, 0)),
            pl.BlockSpec((E, NVARS), lambda s: (0, 0)),
        ],
        out_shape=[
            jax.ShapeDtypeStruct((B, OUT_LEN, NVARS), x.dtype),
            jax.ShapeDtypeStruct((E, NVARS), jnp.float32),
        ],
    )(x, Wg, we_cat)

    gate_mean = gate_sum.T
    return (out, gate_mean)
